# Initial kernel scaffold; baseline (speedup 1.0000x reference)
import jax
import jax.numpy as jnp
from jax.experimental import pallas as pl


def kernel(fourmomenta, scalars, edge_index, batch, is_global, W_lf, b_lf, W_frame, W1, b1, W2, b2, W3, b3):
    raise NotImplementedError("write your pallas kernel here")


# speedup vs baseline: 4.8810x; 4.8810x over previous
import functools

import jax
import jax.numpy as jnp
from jax import lax
from jax.experimental import pallas as pl
from jax.experimental.pallas import tpu as pltpu
from jax.experimental.pallas import tpu_sc as plsc

N = 100000
G = 1024
E = 1600000

BN = 1000
NSTEPS = N // BN

LANES = 128
ROWS_PER_BLK = 8
NSUB = 16
NCORE = 2
EDGE_ROWS = 12544
E_PAD = EDGE_ROWS * LANES
ROWS_PER_TILE = EDGE_ROWS // NSUB
BLKS_PER_TILE = ROWS_PER_TILE // ROWS_PER_BLK
ACC_ROWS = 100352
ROWS_PER_TILE_ACC = ACC_ROWS // NSUB
TRASH = N


def _tc1_body(scal, fm, wlf, blf, wfr, w1a, w1be, w1b, b1r,
              h0o, h1o, h2o, h3o):
    x = jnp.concatenate([scal[...], fm[...]], axis=1)
    xt = jnp.maximum(
        jnp.dot(x, wlf[...], preferred_element_type=jnp.float32) + blf[...], 0.0)
    f = jnp.dot(x, wfr[...], preferred_element_type=jnp.float32)
    pos = fm[...][:, 0:4]
    g = f * jnp.concatenate([pos, pos, pos, pos], axis=1)
    pre = (jnp.dot(xt, w1a[...], preferred_element_type=jnp.float32)
           + jnp.dot(g, w1be[...], preferred_element_type=jnp.float32)
           + jnp.dot(pos, w1b[...], preferred_element_type=jnp.float32)
           + b1r[...])
    h = jnp.maximum(pre, 0.0)
    h0o[...] = h[:, 0:16]
    h1o[...] = h[:, 16:32]
    h2o[...] = h[:, 32:48]
    h3o[...] = h[:, 48:64]


def _tc2_body(h0, h1, h2, h3, m0, m1, m2, m3, batchr,
              w2a, w2b, b2r, w3c, b30, score_o, acc):
    i = pl.program_id(0)
    h = jnp.concatenate([h0[...], h1[...], h2[...], h3[...]], axis=1)
    m = jnp.concatenate([m0[...], m1[...], m2[...], m3[...]], axis=1)
    hh = jnp.maximum(
        jnp.dot(h, w2a[...], preferred_element_type=jnp.float32)
        + jnp.dot(m, w2b[...], preferred_element_type=jnp.float32)
        + b2r[...], 0.0)
    s = jnp.dot(hh, w3c[...], preferred_element_type=jnp.float32) + b30[...]
    onehot = (batchr[...] == lax.broadcasted_iota(jnp.int32, (BN, G), 1)
              ).astype(jnp.float32)
    st = jnp.concatenate([s, jnp.ones_like(s)], axis=1)
    part = lax.dot_general(st, onehot, (((0,), (0,)), ((), ())),
                           preferred_element_type=jnp.float32)

    @pl.when(i == 0)
    def _():
        acc[...] = jnp.zeros_like(acc)

    acc[...] += part

    @pl.when(i == NSTEPS - 1)
    def _():
        score_o[...] = acc[0:1, :] / jnp.maximum(acc[1:2, :], 1.0)


def _sc_edge_body(h0, h1, h2, h3, srcr, dstr, zrows,
                  m0, m1, m2, m3, acc, srcb, dstb, rows, sem):
    cid = lax.axis_index("c")
    sid = lax.axis_index("s")
    hs = [h0, h1, h2, h3]
    ms = [m0, m1, m2, m3]
    edge_base = sid * ROWS_PER_TILE
    acc_base = sid * ROWS_PER_TILE_ACC

    for c in range(NCORE):
        @pl.when(cid == c)
        def _(c=c):
            for p in range(2):
                k = 2 * c + p
                pltpu.sync_copy(zrows, acc.at[pl.ds(acc_base, ROWS_PER_TILE_ACC)])
                plsc.subcore_barrier()

                def blk(b, carry, k=k):
                    r0 = edge_base + b * ROWS_PER_BLK
                    pltpu.sync_copy(srcr.at[pl.ds(r0, ROWS_PER_BLK)], srcb)
                    pltpu.sync_copy(dstr.at[pl.ds(r0, ROWS_PER_BLK)], dstb)
                    descs = [
                        pltpu.async_copy(hs[k].at[srcb.at[j]], rows.at[j], sem)
                        for j in range(ROWS_PER_BLK)
                    ]
                    for d in descs:
                        d.wait()
                    for j in range(ROWS_PER_BLK):
                        pltpu.sync_copy(rows.at[j], acc.at[dstb.at[j]], add=True)
                    return carry

                lax.fori_loop(0, BLKS_PER_TILE, blk, 0)
                plsc.subcore_barrier()
                pltpu.sync_copy(acc.at[pl.ds(acc_base, ROWS_PER_TILE_ACC)],
                                ms[k].at[pl.ds(acc_base, ROWS_PER_TILE_ACC)])
                plsc.subcore_barrier()


def _make_sc_edge():
    mesh = plsc.VectorSubcoreMesh(core_axis_name="c", subcore_axis_name="s")
    return pl.kernel(
        _sc_edge_body,
        out_type=[jax.ShapeDtypeStruct((ACC_ROWS, 16), jnp.float32)
                  for _ in range(4)],
        mesh=mesh,
        scratch_types=[
            pltpu.VMEM_SHARED((ACC_ROWS, 16), jnp.float32),
            pltpu.VMEM((ROWS_PER_BLK, LANES), jnp.int32),
            pltpu.VMEM((ROWS_PER_BLK, LANES), jnp.int32),
            pltpu.VMEM((ROWS_PER_BLK, LANES, 16), jnp.float32),
            pltpu.SemaphoreType.DMA,
        ],
        compiler_params=pltpu.CompilerParams(use_tc_tiling_on_sc=False),
    )


def kernel(fourmomenta, scalars, edge_index, batch, is_global,
           W_lf, b_lf, W_frame, W1, b1, W2, b2, W3, b3):
    f32 = jnp.float32
    fm_flat = fourmomenta.reshape(N, 16).astype(f32)
    scal = scalars.astype(f32)

    W1a = W1[:64]
    W1b = W1[64:68]
    W1be = jnp.repeat(W1b, 4, axis=0)
    full = lambda shape: pl.BlockSpec(shape, lambda i: (0, 0))
    hslices = pl.pallas_call(
        _tc1_body,
        grid=(NSTEPS,),
        in_specs=[
            pl.BlockSpec((BN, 16), lambda i: (i, 0)),
            pl.BlockSpec((BN, 16), lambda i: (i, 0)),
            full((32, 64)), full((1, 64)), full((32, 16)),
            full((64, 64)), full((16, 64)), full((4, 64)), full((1, 64)),
        ],
        out_specs=[pl.BlockSpec((BN, 16), lambda i: (i, 0))] * 4,
        out_shape=[jax.ShapeDtypeStruct((N, 16), f32)] * 4,
    )(scal, fm_flat, W_lf, b_lf.reshape(1, 64), W_frame,
      W1a, W1be, W1b, b1.reshape(1, 64))

    src = edge_index[0].astype(jnp.int32)
    dst = edge_index[1].astype(jnp.int32)
    pad = E_PAD - E
    src_p = jnp.concatenate([src, jnp.zeros((pad,), jnp.int32)]).reshape(
        EDGE_ROWS, LANES)
    dst_p = jnp.concatenate([dst, jnp.full((pad,), TRASH, jnp.int32)]).reshape(
        EDGE_ROWS, LANES)
    zrows = jnp.zeros((ROWS_PER_TILE_ACC, 16), f32)
    mslices = _make_sc_edge()(hslices[0], hslices[1], hslices[2], hslices[3],
                              src_p, dst_p, zrows)
    mslices = [m[:N] for m in mslices]

    W2a = W2[:64]
    W2b = W2[64:]
    w3c = W3[:, 0:1]
    b30 = b3[0].reshape(1, 1)
    score2d = pl.pallas_call(
        _tc2_body,
        grid=(NSTEPS,),
        in_specs=(
            [pl.BlockSpec((BN, 16), lambda i: (i, 0))] * 8
            + [pl.BlockSpec((BN, 1), lambda i: (i, 0)),
               full((64, 64)), full((64, 64)), full((1, 64)),
               full((64, 1)), full((1, 1))]
        ),
        out_specs=pl.BlockSpec((1, G), lambda i: (0, 0)),
        out_shape=jax.ShapeDtypeStruct((1, G), f32),
        scratch_shapes=[pltpu.VMEM((2, G), f32)],
    )(hslices[0], hslices[1], hslices[2], hslices[3],
      mslices[0], mslices[1], mslices[2], mslices[3],
      batch.astype(jnp.int32).reshape(N, 1),
      W2a, W2b, b2.reshape(1, 64), w3c, b30)
    return score2d.reshape(G)


# speedup vs baseline: 6.0046x; 1.2302x over previous
import functools

import jax
import jax.numpy as jnp
from jax import lax
from jax.experimental import pallas as pl
from jax.experimental.pallas import tpu as pltpu
from jax.experimental.pallas import tpu_sc as plsc

N = 100000
G = 1024
E = 1600000

BN1 = 4000
NSTEPS1 = N // BN1
BN2 = 2000
NSTEPS2 = N // BN2

LANES = 100
RPB = 4
NSUB = 16
NCORE = 2
EDGE_ROWS = E // LANES
ROWS_PER_TILE = EDGE_ROWS // NSUB
BLKS_PER_TILE = ROWS_PER_TILE // RPB
ACC_ROWS = N
ACC_PER_TILE = ACC_ROWS // NSUB


def _tc1_body(scal, fm, wlf, blf, wfr, w1a, w1be, w1b, b1r,
              h0o, h1o, h2o, h3o):
    x = jnp.concatenate([scal[...], fm[...]], axis=1)
    xt = jnp.maximum(
        jnp.dot(x, wlf[...], preferred_element_type=jnp.float32) + blf[...], 0.0)
    f = jnp.dot(x, wfr[...], preferred_element_type=jnp.float32)
    pos = fm[...][:, 0:4]
    g = f * jnp.concatenate([pos, pos, pos, pos], axis=1)
    pre = (jnp.dot(xt, w1a[...], preferred_element_type=jnp.float32)
           + jnp.dot(g, w1be[...], preferred_element_type=jnp.float32)
           + jnp.dot(pos, w1b[...], preferred_element_type=jnp.float32)
           + b1r[...])
    h = jnp.maximum(pre, 0.0)
    h0o[...] = h[:, 0:16]
    h1o[...] = h[:, 16:32]
    h2o[...] = h[:, 32:48]
    h3o[...] = h[:, 48:64]


def _tc2_body(h0, h1, h2, h3, m0, m1, m2, m3, batchr,
              w2a, w2b, b2r, w3c, b30, score_o, acc):
    i = pl.program_id(0)
    h = jnp.concatenate([h0[...], h1[...], h2[...], h3[...]], axis=1)
    m = jnp.concatenate([m0[...], m1[...], m2[...], m3[...]], axis=1)
    hh = jnp.maximum(
        jnp.dot(h, w2a[...], preferred_element_type=jnp.float32)
        + jnp.dot(m, w2b[...], preferred_element_type=jnp.float32)
        + b2r[...], 0.0)
    s = jnp.dot(hh, w3c[...], preferred_element_type=jnp.float32) + b30[...]
    onehot = (batchr[...] == lax.broadcasted_iota(jnp.int32, (BN2, G), 1)
              ).astype(jnp.float32)
    st = jnp.concatenate([s, jnp.ones_like(s)], axis=1)
    part = lax.dot_general(st, onehot, (((0,), (0,)), ((), ())),
                           preferred_element_type=jnp.float32)

    @pl.when(i == 0)
    def _():
        acc[...] = jnp.zeros_like(acc)

    acc[...] += part

    @pl.when(i == NSTEPS2 - 1)
    def _():
        score_o[...] = acc[0:1, :] / jnp.maximum(acc[1:2, :], 1.0)


def _sc_edge_body(h0, h1, h2, h3, ei, zrows, m0, m1, m2, m3,
                  acc, srcb, dstb, rows, es0, es1, gsem, ss0, ss1):
    cid = lax.axis_index("c")
    sid = lax.axis_index("s")
    hs = [h0, h1, h2, h3]
    ms = [m0, m1, m2, m3]
    esems = (es0, es1)
    ssems = (ss0, ss1)
    srcr = ei.at[0]
    dstr = ei.at[1]
    base = sid * ROWS_PER_TILE
    acc_lo = sid * ACC_PER_TILE

    for c in range(NCORE):
        @pl.when(cid == c)
        def _(c=c):
            for p in range(2):
                k = 2 * c + p
                hk = hs[k]
                mk = ms[k]
                pltpu.sync_copy(zrows, acc.at[pl.ds(acc_lo, ACC_PER_TILE)])
                plsc.subcore_barrier()

                pltpu.async_copy(srcr.at[pl.ds(base, RPB)], srcb.at[0], es0)
                pltpu.async_copy(dstr.at[pl.ds(base, RPB)], dstb.at[0], es0)

                def body(t, carry, hk=hk):
                    for u in range(2):
                        b = 2 * t + u
                        o = 1 - u
                        r_next = base + (b + 1) * RPB
                        pltpu.make_async_copy(
                            srcr.at[pl.ds(0, RPB)], srcb.at[u], esems[u]).wait()
                        pltpu.make_async_copy(
                            srcr.at[pl.ds(0, RPB)], dstb.at[u], esems[u]).wait()

                        def drain_prev(o=o, hk=hk):
                            for j in range(RPB):
                                pltpu.make_async_copy(
                                    hk.at[pl.ds(0, LANES)], rows.at[o, j],
                                    ssems[o]).wait()
                        if u == 0:
                            pl.when(t > 0)(drain_prev)
                        else:
                            drain_prev()

                        if u == 0:
                            pltpu.async_copy(
                                srcr.at[pl.ds(r_next, RPB)], srcb.at[1], es1)
                            pltpu.async_copy(
                                dstr.at[pl.ds(r_next, RPB)], dstb.at[1], es1)
                        else:
                            @pl.when(t < BLKS_PER_TILE // 2 - 1)
                            def _():
                                pltpu.async_copy(
                                    srcr.at[pl.ds(r_next, RPB)], srcb.at[0], es0)
                                pltpu.async_copy(
                                    dstr.at[pl.ds(r_next, RPB)], dstb.at[0], es0)

                        gds = [
                            pltpu.async_copy(hk.at[srcb.at[u, j]],
                                             rows.at[u, j], gsem)
                            for j in range(RPB)
                        ]
                        for d in gds:
                            d.wait()
                        for j in range(RPB):
                            pltpu.async_copy(rows.at[u, j],
                                             acc.at[dstb.at[u, j]],
                                             ssems[u], add=True)
                    return carry

                lax.fori_loop(0, BLKS_PER_TILE // 2, body, 0)
                for j in range(RPB):
                    pltpu.make_async_copy(
                        hk.at[pl.ds(0, LANES)], rows.at[1, j],
                        ssems[1]).wait()
                plsc.subcore_barrier()
                pltpu.sync_copy(acc.at[pl.ds(acc_lo, ACC_PER_TILE)],
                                mk.at[pl.ds(acc_lo, ACC_PER_TILE)])
                plsc.subcore_barrier()


def _make_sc_edge():
    mesh = plsc.VectorSubcoreMesh(core_axis_name="c", subcore_axis_name="s")
    return pl.kernel(
        _sc_edge_body,
        out_type=[jax.ShapeDtypeStruct((N, 16), jnp.float32)
                  for _ in range(4)],
        mesh=mesh,
        scratch_types=[
            pltpu.VMEM_SHARED((ACC_ROWS, 16), jnp.float32),
            pltpu.VMEM((2, RPB, LANES), jnp.int32),
            pltpu.VMEM((2, RPB, LANES), jnp.int32),
            pltpu.VMEM((2, RPB, LANES, 16), jnp.float32),
            pltpu.SemaphoreType.DMA,
            pltpu.SemaphoreType.DMA,
            pltpu.SemaphoreType.DMA,
            pltpu.SemaphoreType.DMA,
            pltpu.SemaphoreType.DMA,
        ],
        compiler_params=pltpu.CompilerParams(use_tc_tiling_on_sc=False),
    )


def kernel(fourmomenta, scalars, edge_index, batch, is_global,
           W_lf, b_lf, W_frame, W1, b1, W2, b2, W3, b3):
    f32 = jnp.float32
    fm_flat = fourmomenta.reshape(N, 16).astype(f32)
    scal = scalars.astype(f32)

    W1a = W1[:64]
    W1b = W1[64:68]
    W1be = jnp.repeat(W1b, 4, axis=0)
    full = lambda shape: pl.BlockSpec(shape, lambda i: (0, 0))
    hslices = pl.pallas_call(
        _tc1_body,
        grid=(NSTEPS1,),
        in_specs=[
            pl.BlockSpec((BN1, 16), lambda i: (i, 0)),
            pl.BlockSpec((BN1, 16), lambda i: (i, 0)),
            full((32, 64)), full((1, 64)), full((32, 16)),
            full((64, 64)), full((16, 64)), full((4, 64)), full((1, 64)),
        ],
        out_specs=[pl.BlockSpec((BN1, 16), lambda i: (i, 0))] * 4,
        out_shape=[jax.ShapeDtypeStruct((N, 16), f32)] * 4,
    )(scal, fm_flat, W_lf, b_lf.reshape(1, 64), W_frame,
      W1a, W1be, W1b, b1.reshape(1, 64))

    ei3 = edge_index.astype(jnp.int32).reshape(2, EDGE_ROWS, LANES)
    zrows = jnp.zeros((ACC_PER_TILE, 16), f32)
    mslices = _make_sc_edge()(hslices[0], hslices[1], hslices[2], hslices[3],
                              ei3, zrows)

    W2a = W2[:64]
    W2b = W2[64:]
    w3c = W3[:, 0:1]
    b30 = b3[0].reshape(1, 1)
    score2d = pl.pallas_call(
        _tc2_body,
        grid=(NSTEPS2,),
        in_specs=(
            [pl.BlockSpec((BN2, 16), lambda i: (i, 0))] * 8
            + [pl.BlockSpec((BN2, 1), lambda i: (i, 0)),
               full((64, 64)), full((64, 64)), full((1, 64)),
               full((64, 1)), full((1, 1))]
        ),
        out_specs=pl.BlockSpec((1, G), lambda i: (0, 0)),
        out_shape=jax.ShapeDtypeStruct((1, G), f32),
        scratch_shapes=[pltpu.VMEM((2, G), f32)],
    )(hslices[0], hslices[1], hslices[2], hslices[3],
      mslices[0], mslices[1], mslices[2], mslices[3],
      batch.astype(jnp.int32).reshape(N, 1),
      W2a, W2b, b2.reshape(1, 64), w3c, b30)
    return score2d.reshape(G)


# speedup vs baseline: 6.7297x; 1.1208x over previous
import functools

import jax
import jax.numpy as jnp
from jax import lax
from jax.experimental import pallas as pl
from jax.experimental.pallas import tpu as pltpu
from jax.experimental.pallas import tpu_sc as plsc

N = 100000
G = 1024
E = 1600000

BN1 = 4000
NSTEPS1 = N // BN1
BN2 = 2000
NSTEPS2 = N // BN2

LANES = 125
RPB = 4
NSUB = 16
NCORE = 2
EDGE_ROWS = E // LANES
ROWS_PER_TILE = EDGE_ROWS // NSUB
BLKS_PER_TILE = ROWS_PER_TILE // RPB
ACC_ROWS = N
ACC_PER_TILE = ACC_ROWS // NSUB


def _tc1_body(scal, fm, wlf, blf, wfr, w1a, w1be, w1b, b1r,
              h0o, h1o, h2o, h3o):
    x = jnp.concatenate([scal[...], fm[...]], axis=1)
    xt = jnp.maximum(
        jnp.dot(x, wlf[...], preferred_element_type=jnp.float32) + blf[...], 0.0)
    f = jnp.dot(x, wfr[...], preferred_element_type=jnp.float32)
    pos = fm[...][:, 0:4]
    g = f * jnp.concatenate([pos, pos, pos, pos], axis=1)
    pre = (jnp.dot(xt, w1a[...], preferred_element_type=jnp.float32)
           + jnp.dot(g, w1be[...], preferred_element_type=jnp.float32)
           + jnp.dot(pos, w1b[...], preferred_element_type=jnp.float32)
           + b1r[...])
    h = jnp.maximum(pre, 0.0)
    h0o[...] = h[:, 0:16]
    h1o[...] = h[:, 16:32]
    h2o[...] = h[:, 32:48]
    h3o[...] = h[:, 48:64]


def _tc2_body(h0, h1, h2, h3, m0, m1, m2, m3, batchr,
              w2a, w2b, b2r, w3c, b30, score_o, acc):
    i = pl.program_id(0)
    h = jnp.concatenate([h0[...], h1[...], h2[...], h3[...]], axis=1)
    m = jnp.concatenate([m0[...], m1[...], m2[...], m3[...]], axis=1)
    hh = jnp.maximum(
        jnp.dot(h, w2a[...], preferred_element_type=jnp.float32)
        + jnp.dot(m, w2b[...], preferred_element_type=jnp.float32)
        + b2r[...], 0.0)
    s = jnp.dot(hh, w3c[...], preferred_element_type=jnp.float32) + b30[...]
    onehot = (batchr[...] == lax.broadcasted_iota(jnp.int32, (BN2, G), 1)
              ).astype(jnp.float32)
    st = jnp.concatenate([s, jnp.ones_like(s)], axis=1)
    part = lax.dot_general(st, onehot, (((0,), (0,)), ((), ())),
                           preferred_element_type=jnp.float32)

    @pl.when(i == 0)
    def _():
        acc[...] = jnp.zeros_like(acc)

    acc[...] += part

    @pl.when(i == NSTEPS2 - 1)
    def _():
        score_o[...] = acc[0:1, :] / jnp.maximum(acc[1:2, :], 1.0)


def _sc_edge_body(h0, h1, h2, h3, ei, zrows, m0, m1, m2, m3,
                  acc, srcb, dstb, rows, es0, es1, gsem, ss0, ss1):
    cid = lax.axis_index("c")
    sid = lax.axis_index("s")
    hs = [h0, h1, h2, h3]
    ms = [m0, m1, m2, m3]
    esems = (es0, es1)
    ssems = (ss0, ss1)
    srcr = ei.at[0]
    dstr = ei.at[1]
    base = sid * ROWS_PER_TILE
    acc_lo = sid * ACC_PER_TILE

    for c in range(NCORE):
        @pl.when(cid == c)
        def _(c=c):
            for p in range(2):
                k = 2 * c + p
                hk = hs[k]
                mk = ms[k]
                pltpu.sync_copy(zrows, acc.at[pl.ds(acc_lo, ACC_PER_TILE)])
                plsc.subcore_barrier()

                pltpu.async_copy(srcr.at[pl.ds(base, RPB)], srcb.at[0], es0)
                pltpu.async_copy(dstr.at[pl.ds(base, RPB)], dstb.at[0], es0)

                def body(t, carry, hk=hk):
                    for u in range(2):
                        b = 2 * t + u
                        o = 1 - u
                        r_next = base + (b + 1) * RPB
                        pltpu.make_async_copy(
                            srcr.at[pl.ds(0, RPB)], srcb.at[u], esems[u]).wait()
                        pltpu.make_async_copy(
                            srcr.at[pl.ds(0, RPB)], dstb.at[u], esems[u]).wait()

                        def drain_prev(o=o, hk=hk):
                            for j in range(RPB):
                                pltpu.make_async_copy(
                                    hk.at[pl.ds(0, LANES)], rows.at[o, j],
                                    ssems[o]).wait()
                        if u == 0:
                            pl.when(t > 0)(drain_prev)
                        else:
                            drain_prev()

                        if u == 0:
                            pltpu.async_copy(
                                srcr.at[pl.ds(r_next, RPB)], srcb.at[1], es1)
                            pltpu.async_copy(
                                dstr.at[pl.ds(r_next, RPB)], dstb.at[1], es1)
                        else:
                            @pl.when(t < BLKS_PER_TILE // 2 - 1)
                            def _():
                                pltpu.async_copy(
                                    srcr.at[pl.ds(r_next, RPB)], srcb.at[0], es0)
                                pltpu.async_copy(
                                    dstr.at[pl.ds(r_next, RPB)], dstb.at[0], es0)

                        gds = [
                            pltpu.async_copy(hk.at[srcb.at[u, j]],
                                             rows.at[u, j], gsem)
                            for j in range(RPB)
                        ]
                        for d in gds:
                            d.wait()
                        for j in range(RPB):
                            pltpu.async_copy(rows.at[u, j],
                                             acc.at[dstb.at[u, j]],
                                             ssems[u], add=True)
                    return carry

                lax.fori_loop(0, BLKS_PER_TILE // 2, body, 0)
                for j in range(RPB):
                    pltpu.make_async_copy(
                        hk.at[pl.ds(0, LANES)], rows.at[1, j],
                        ssems[1]).wait()
                plsc.subcore_barrier()
                pltpu.sync_copy(acc.at[pl.ds(acc_lo, ACC_PER_TILE)],
                                mk.at[pl.ds(acc_lo, ACC_PER_TILE)])
                plsc.subcore_barrier()


def _make_sc_edge():
    mesh = plsc.VectorSubcoreMesh(core_axis_name="c", subcore_axis_name="s")
    return pl.kernel(
        _sc_edge_body,
        out_type=[jax.ShapeDtypeStruct((N, 16), jnp.float32)
                  for _ in range(4)],
        mesh=mesh,
        scratch_types=[
            pltpu.VMEM_SHARED((ACC_ROWS, 16), jnp.float32),
            pltpu.VMEM((2, RPB, LANES), jnp.int32),
            pltpu.VMEM((2, RPB, LANES), jnp.int32),
            pltpu.VMEM((2, RPB, LANES, 16), jnp.float32),
            pltpu.SemaphoreType.DMA,
            pltpu.SemaphoreType.DMA,
            pltpu.SemaphoreType.DMA,
            pltpu.SemaphoreType.DMA,
            pltpu.SemaphoreType.DMA,
        ],
        compiler_params=pltpu.CompilerParams(use_tc_tiling_on_sc=False),
    )


def kernel(fourmomenta, scalars, edge_index, batch, is_global,
           W_lf, b_lf, W_frame, W1, b1, W2, b2, W3, b3):
    f32 = jnp.float32
    fm_flat = fourmomenta.reshape(N, 16).astype(f32)
    scal = scalars.astype(f32)

    W1a = W1[:64]
    W1b = W1[64:68]
    W1be = jnp.repeat(W1b, 4, axis=0)
    full = lambda shape: pl.BlockSpec(shape, lambda i: (0, 0))
    hslices = pl.pallas_call(
        _tc1_body,
        grid=(NSTEPS1,),
        in_specs=[
            pl.BlockSpec((BN1, 16), lambda i: (i, 0)),
            pl.BlockSpec((BN1, 16), lambda i: (i, 0)),
            full((32, 64)), full((1, 64)), full((32, 16)),
            full((64, 64)), full((16, 64)), full((4, 64)), full((1, 64)),
        ],
        out_specs=[pl.BlockSpec((BN1, 16), lambda i: (i, 0))] * 4,
        out_shape=[jax.ShapeDtypeStruct((N, 16), f32)] * 4,
    )(scal, fm_flat, W_lf, b_lf.reshape(1, 64), W_frame,
      W1a, W1be, W1b, b1.reshape(1, 64))

    ei3 = edge_index.astype(jnp.int32).reshape(2, EDGE_ROWS, LANES)
    zrows = jnp.zeros((ACC_PER_TILE, 16), f32)
    mslices = _make_sc_edge()(hslices[0], hslices[1], hslices[2], hslices[3],
                              ei3, zrows)

    W2a = W2[:64]
    W2b = W2[64:]
    w3c = W3[:, 0:1]
    b30 = b3[0].reshape(1, 1)
    score2d = pl.pallas_call(
        _tc2_body,
        grid=(NSTEPS2,),
        in_specs=(
            [pl.BlockSpec((BN2, 16), lambda i: (i, 0))] * 8
            + [pl.BlockSpec((BN2, 1), lambda i: (i, 0)),
               full((64, 64)), full((64, 64)), full((1, 64)),
               full((64, 1)), full((1, 1))]
        ),
        out_specs=pl.BlockSpec((1, G), lambda i: (0, 0)),
        out_shape=jax.ShapeDtypeStruct((1, G), f32),
        scratch_shapes=[pltpu.VMEM((2, G), f32)],
    )(hslices[0], hslices[1], hslices[2], hslices[3],
      mslices[0], mslices[1], mslices[2], mslices[3],
      batch.astype(jnp.int32).reshape(N, 1),
      W2a, W2b, b2.reshape(1, 64), w3c, b30)
    return score2d.reshape(G)
